# baseline (speedup 1.0000x reference)
import functools
import itertools

import jax
import jax.numpy as jnp
import numpy as np
from jax.experimental import pallas as pl

_COMBOS = list(itertools.combinations(range(4), 2))


def _build_patterns():
    pats = []
    for vs in itertools.product(range(6), repeat=4):
        cols = [0, 0, 0, 0]
        for v in vs:
            for c in _COMBOS[v]:
                cols[c] += 1
        if all(cc == 2 for cc in cols):
            bits = 0
            for r, v in enumerate(vs):
                for c in _COMBOS[v]:
                    bits |= 1 << (4 * r + c)
            pats.append((*vs, bits))
    assert len(pats) == 90
    return pats


_PATTERNS = _build_patterns()

_LPERM = np.zeros((32, 32), dtype=np.float32)
for _k in range(32):
    _LPERM[_k, 4 * (_k % 8) + (_k // 8)] = 1.0
_LEXP = np.zeros((32, 8), dtype=np.float32)
for _s in range(32):
    _LEXP[_s, _s // 4] = 1.0

_CHAIN = 30


def _process_chunk(x, lperm, lexp):
    w = x.shape[1]
    a = jnp.abs(x).astype(jnp.bfloat16)
    g = jax.lax.dot(
        lperm, a, preferred_element_type=jnp.float32
    )

    t = {}
    for r in range(4):
        ar = g[8 * r : 8 * (r + 1), :]
        z = {s: ar + jnp.roll(ar, -s, axis=1) for s in (1, 2, 3)}
        for v, (c1, c2) in enumerate(_COMBOS):
            zz = z[c2 - c1]
            t[(r, v)] = zz if c1 == 0 else jnp.roll(zz, -c1, axis=1)

    half23 = {}
    for _, _, v2, v3, _ in _PATTERNS:
        if (v2, v3) not in half23:
            half23[(v2, v3)] = t[(2, v2)] + t[(3, v3)]

    chains = []
    for c0 in range(0, 90, _CHAIN):
        best_s = None
        best_bits = None
        half01 = {}
        for v0, v1, v2, v3, bits in _PATTERNS[c0 : c0 + _CHAIN]:
            if (v0, v1) not in half01:
                half01[(v0, v1)] = t[(0, v0)] + t[(1, v1)]
            s = half01[(v0, v1)] + half23[(v2, v3)]
            if best_s is None:
                best_s = s
                best_bits = jnp.full(s.shape, bits, dtype=jnp.int32)
            else:
                upd = s > best_s
                best_s = jnp.where(upd, s, best_s)
                best_bits = jnp.where(upd, jnp.int32(bits), best_bits)
        chains.append((best_s, best_bits))
    best_s, best_bits = chains[0]
    for cs, cb in chains[1:]:
        upd = cs > best_s
        best_s = jnp.where(upd, cs, best_s)
        best_bits = jnp.where(upd, cb, best_bits)

    lane8 = jax.lax.broadcasted_iota(jnp.int32, (8, w), 1) % 4
    bz = jnp.where(lane8 == 0, best_bits, 0)
    bb = bz | jnp.roll(bz, 1, axis=1) | jnp.roll(bz, 2, axis=1) | jnp.roll(bz, 3, axis=1)
    ulo = jax.lax.dot(
        lexp, (bb & 255).astype(jnp.bfloat16), preferred_element_type=jnp.float32
    )
    uhi = jax.lax.dot(
        lexp, (bb >> 8).astype(jnp.bfloat16), preferred_element_type=jnp.float32
    )
    ui = ulo.astype(jnp.int32) | (uhi.astype(jnp.int32) << 8)
    sub4 = jax.lax.broadcasted_iota(jnp.int32, x.shape, 0) % 4
    lane4 = jax.lax.broadcasted_iota(jnp.int32, x.shape, 1) % 4
    mbit = (ui >> (4 * sub4 + lane4)) & 1
    return x * mbit.astype(jnp.float32), mbit > 0


def _tile_kernel(x_ref, lperm_ref, lexp_ref, sparse_ref, mask_ref):
    tn = x_ref.shape[1]
    lperm = lperm_ref[...]
    lexp = lexp_ref[...]
    for c in range(0, tn, 128):
        xc = x_ref[:, c : c + 128]
        sp, mk = _process_chunk(xc, lperm, lexp)
        sparse_ref[:, c : c + 128] = sp
        mask_ref[:, c : c + 128] = mk


@functools.partial(jax.jit, static_argnames=("tn",))
def _run(x, tn):
    m, k = x.shape
    grid = (m // 32, k // tn)
    sparse, mask = pl.pallas_call(
        _tile_kernel,
        grid=grid,
        in_specs=[
            pl.BlockSpec((32, tn), lambda i, j: (i, j)),
            pl.BlockSpec((32, 32), lambda i, j: (0, 0)),
            pl.BlockSpec((32, 8), lambda i, j: (0, 0)),
        ],
        out_specs=[
            pl.BlockSpec((32, tn), lambda i, j: (i, j)),
            pl.BlockSpec((32, tn), lambda i, j: (i, j)),
        ],
        out_shape=[
            jax.ShapeDtypeStruct((m, k), jnp.float32),
            jax.ShapeDtypeStruct((m, k), jnp.bool_),
        ],
    )(
        x,
        jnp.asarray(_LPERM, dtype=jnp.bfloat16),
        jnp.asarray(_LEXP, dtype=jnp.bfloat16),
    )
    return sparse, mask


def kernel(x, mask_pattern):
    del mask_pattern
    return _run(x, 512)


# speedup vs baseline: 1.6392x; 1.6392x over previous
import functools
import itertools

import jax
import jax.numpy as jnp
import numpy as np
from jax.experimental import pallas as pl

_COMBOS = list(itertools.combinations(range(4), 2))


def _build_patterns():
    pats = []
    for vs in itertools.product(range(6), repeat=4):
        cols = [0, 0, 0, 0]
        for v in vs:
            for c in _COMBOS[v]:
                cols[c] += 1
        if all(cc == 2 for cc in cols):
            bits = 0
            for r, v in enumerate(vs):
                for c in _COMBOS[v]:
                    bits |= 1 << (4 * r + c)
            pats.append((*vs, bits))
    assert len(pats) == 90
    return pats


_PATTERNS = _build_patterns()

_LPERM = np.zeros((32, 32), dtype=np.float32)
for _k in range(32):
    _LPERM[_k, 4 * (_k % 8) + (_k // 8)] = 1.0
_LEXP = np.zeros((32, 8), dtype=np.float32)
for _s in range(32):
    _LEXP[_s, _s // 4] = 1.0

_CHAIN = 30


def _process_chunk(x, lperm, lexp):
    w = x.shape[1]
    a = jnp.abs(x).astype(jnp.bfloat16)
    g = jax.lax.dot(
        lperm, a, preferred_element_type=jnp.float32
    )

    t = {}
    for r in range(4):
        ar = g[8 * r : 8 * (r + 1), :]
        z = {s: ar + jnp.roll(ar, -s, axis=1) for s in (1, 2, 3)}
        for v, (c1, c2) in enumerate(_COMBOS):
            zz = z[c2 - c1]
            t[(r, v)] = zz if c1 == 0 else jnp.roll(zz, -c1, axis=1)

    half23 = {}
    for _, _, v2, v3, _ in _PATTERNS:
        if (v2, v3) not in half23:
            half23[(v2, v3)] = t[(2, v2)] + t[(3, v3)]

    chains = []
    for c0 in range(0, 90, _CHAIN):
        best_s = None
        best_bits = None
        half01 = {}
        for v0, v1, v2, v3, bits in _PATTERNS[c0 : c0 + _CHAIN]:
            if (v0, v1) not in half01:
                half01[(v0, v1)] = t[(0, v0)] + t[(1, v1)]
            s = half01[(v0, v1)] + half23[(v2, v3)]
            if best_s is None:
                best_s = s
                best_bits = jnp.full(s.shape, bits, dtype=jnp.int32)
            else:
                upd = s > best_s
                best_s = jnp.where(upd, s, best_s)
                best_bits = jnp.where(upd, jnp.int32(bits), best_bits)
        chains.append((best_s, best_bits))
    best_s, best_bits = chains[0]
    for cs, cb in chains[1:]:
        upd = cs > best_s
        best_s = jnp.where(upd, cs, best_s)
        best_bits = jnp.where(upd, cb, best_bits)

    lane8 = jax.lax.broadcasted_iota(jnp.int32, (8, w), 1) % 4
    bz = jnp.where(lane8 == 0, best_bits, 0)
    bb = bz | jnp.roll(bz, 1, axis=1) | jnp.roll(bz, 2, axis=1) | jnp.roll(bz, 3, axis=1)
    ulo = jax.lax.dot(
        lexp, (bb & 255).astype(jnp.bfloat16), preferred_element_type=jnp.float32
    )
    uhi = jax.lax.dot(
        lexp, (bb >> 8).astype(jnp.bfloat16), preferred_element_type=jnp.float32
    )
    ui = ulo.astype(jnp.int32) | (uhi.astype(jnp.int32) << 8)
    sub4 = jax.lax.broadcasted_iota(jnp.int32, x.shape, 0) % 4
    lane4 = jax.lax.broadcasted_iota(jnp.int32, x.shape, 1) % 4
    mbit = (ui >> (4 * sub4 + lane4)) & 1
    return x * mbit.astype(jnp.float32), mbit > 0


def _tile_kernel(x_ref, lperm_ref, lexp_ref, sparse_ref, mask_ref):
    tn = x_ref.shape[1]
    lperm = lperm_ref[...]
    lexp = lexp_ref[...]
    for c in range(0, tn, 256):
        xc = x_ref[:, c : c + 256]
        sp, mk = _process_chunk(xc, lperm, lexp)
        sparse_ref[:, c : c + 256] = sp
        mask_ref[:, c : c + 256] = mk


@functools.partial(jax.jit, static_argnames=("tn",))
def _run(x, tn):
    m, k = x.shape
    grid = (m // 32, k // tn)
    sparse, mask = pl.pallas_call(
        _tile_kernel,
        grid=grid,
        in_specs=[
            pl.BlockSpec((32, tn), lambda i, j: (i, j)),
            pl.BlockSpec((32, 32), lambda i, j: (0, 0)),
            pl.BlockSpec((32, 8), lambda i, j: (0, 0)),
        ],
        out_specs=[
            pl.BlockSpec((32, tn), lambda i, j: (i, j)),
            pl.BlockSpec((32, tn), lambda i, j: (i, j)),
        ],
        out_shape=[
            jax.ShapeDtypeStruct((m, k), jnp.float32),
            jax.ShapeDtypeStruct((m, k), jnp.bool_),
        ],
    )(
        x,
        jnp.asarray(_LPERM, dtype=jnp.bfloat16),
        jnp.asarray(_LEXP, dtype=jnp.bfloat16),
    )
    return sparse, mask


def kernel(x, mask_pattern):
    del mask_pattern
    return _run(x, 512)
